# baseline (speedup 1.0000x reference)
import functools

import jax
import jax.numpy as jnp
from jax import lax
from jax.experimental import pallas as pl
from jax.experimental.pallas import tpu as pltpu
from jax.experimental.pallas import tpu_sc as plsc

VOCAB = 100000
BATCH = 16384
D = 128
D2 = 400

NC = 2
NS = 16
NW = NC * NS

B_PER_W = BATCH // NW
CHUNK = 32
NCHUNK = B_PER_W // CHUNK
L = 16

BC = 2048
ROWS_LO = D2 // NW
EXTRA = D2 - ROWS_LO * NW


def _sc_kernel(t0_hbm, t1_hbm, t2T_hbm, idxT_hbm, out0_hbm, out1T_hbm,
               row_v, sem_r):
    wid = lax.axis_index("s") * NC + lax.axis_index("c")
    base = wid * B_PER_W

    nrows = lax.select(wid < EXTRA, ROWS_LO + 1, ROWS_LO)
    start = lax.select(wid < EXTRA, (ROWS_LO + 1) * wid,
                       EXTRA + ROWS_LO * wid)
    pltpu.async_copy(t2T_hbm.at[start], row_v, sem_r)

    def part1(idx0_v, idx1_v, r0_v, r1_v, sem_g, sem_w):
        pltpu.sync_copy(idxT_hbm.at[0, pl.ds(base, B_PER_W)], idx0_v)
        pltpu.sync_copy(idxT_hbm.at[1, pl.ds(base, B_PER_W)], idx1_v)

        def load_and_fire(c, p):
            s = pl.ds(c * CHUNK, CHUNK)
            return [
                pltpu.async_copy(t0_hbm.at[idx0_v.at[s]], r0_v.at[p], sem_g),
                pltpu.async_copy(t1_hbm.at[idx1_v.at[s]], r1_v.at[p], sem_g),
            ]

        handles_g = [None, None]
        handles_w = [None, None]
        handles_g[0] = load_and_fire(0, 0)

        for c in range(NCHUNK):
            p = c % 2
            q = 1 - p
            if c + 1 < NCHUNK:
                if handles_w[q] is not None:
                    for h in handles_w[q]:
                        h.wait()
                handles_g[q] = load_and_fire(c + 1, q)
            for h in handles_g[p]:
                h.wait()

            def row_body(r, carry):
                for j in range(D // L):
                    s = pl.ds(j * L, L)
                    r0_v[p, r, s] = r0_v[p, r, s] + r1_v[p, r, s]
                return carry

            lax.fori_loop(0, CHUNK, row_body, 0)

            off = base + c * CHUNK
            handles_w[p] = [
                pltpu.async_copy(r0_v.at[p], out0_hbm.at[pl.ds(off, CHUNK)],
                                 sem_w),
            ]

        for hs in handles_w:
            if hs is not None:
                for h in hs:
                    h.wait()

    pl.run_scoped(
        part1,
        pltpu.VMEM((B_PER_W,), jnp.int32),
        pltpu.VMEM((B_PER_W,), jnp.int32),
        pltpu.VMEM((2, CHUNK, D), jnp.float32),
        pltpu.VMEM((2, CHUNK, D), jnp.float32),
        pltpu.SemaphoreType.DMA,
        pltpu.SemaphoreType.DMA,
    )

    def part2(idx_v, outc_v, sem_w):
        pltpu.sync_copy(idxT_hbm.at[1], idx_v)

        def row_loop(i, carry):
            c = start + i
            pltpu.make_async_copy(t2T_hbm.at[c], row_v, sem_r).wait()
            for bc in range(BATCH // BC):
                p = bc % 2

                @pl.when(i * (BATCH // BC) + bc > 1)
                def _drain():
                    pltpu.make_async_copy(
                        outc_v.at[p],
                        out1T_hbm.at[c, pl.ds(bc * BC, BC)], sem_w).wait()

                def g_loop(g, carry2):
                    for u in range(16):
                        s = pl.ds(bc * BC + (g * 16 + u) * L, L)
                        so = pl.ds((g * 16 + u) * L, L)
                        outc_v[p, so] = plsc.load_gather(row_v, [idx_v[s]])
                    return carry2

                lax.fori_loop(0, BC // (16 * L), g_loop, 0)
                pltpu.async_copy(outc_v.at[p],
                                 out1T_hbm.at[c, pl.ds(bc * BC, BC)], sem_w)

            @pl.when(i + 1 < nrows)
            def _prefetch():
                pltpu.async_copy(t2T_hbm.at[c + 1], row_v, sem_r)

            return carry

        lax.fori_loop(0, nrows, row_loop, 0)
        last = start + nrows - 1
        for bc in range(BATCH // BC - 2, BATCH // BC):
            pltpu.make_async_copy(
                outc_v.at[bc % 2],
                out1T_hbm.at[last, pl.ds(bc * BC, BC)], sem_w).wait()

    pl.run_scoped(
        part2,
        pltpu.VMEM((BATCH,), jnp.int32),
        pltpu.VMEM((2, BC), jnp.float32),
        pltpu.SemaphoreType.DMA,
    )


@jax.jit
def _run(t0, t1, t2T, idxT):
    mesh = plsc.VectorSubcoreMesh(core_axis_name="c", subcore_axis_name="s")
    fn = functools.partial(
        pl.kernel, mesh=mesh,
        compiler_params=pltpu.CompilerParams(needs_layout_passes=False),
        out_type=[
            jax.ShapeDtypeStruct((BATCH, D), jnp.float32),
            jax.ShapeDtypeStruct((D2, BATCH), jnp.float32),
        ],
        scratch_types=[
            pltpu.VMEM((VOCAB,), jnp.float32),
            pltpu.SemaphoreType.DMA,
        ],
    )(_sc_kernel)
    return fn(t0, t1, t2T, idxT)


def kernel(arg0_1, arg1_1, arg2_1, arg3_1):
    t2T = arg2_1.T
    idxT = arg3_1.astype(jnp.int32).T
    out0, out1T = _run(arg0_1, arg1_1, t2T, idxT)
    return (out0, out1T.T)


# speedup vs baseline: 1.2912x; 1.2912x over previous
import functools

import jax
import jax.numpy as jnp
from jax import lax
from jax.experimental import pallas as pl
from jax.experimental.pallas import tpu as pltpu
from jax.experimental.pallas import tpu_sc as plsc

VOCAB = 100000
BATCH = 16384
D = 128
D2 = 400

NC = 2
NS = 16
NW = NC * NS

B_PER_W = BATCH // NW
CHUNK = 128
NCHUNK = B_PER_W // CHUNK
L = 16

BC = 4096
ROWS_LO = D2 // NW
EXTRA = D2 - ROWS_LO * NW


def _sc_kernel(t0_hbm, t1_hbm, t2T_hbm, idxT_hbm, out0_hbm, out1T_hbm):
    wid = lax.axis_index("s") * NC + lax.axis_index("c")
    base = wid * B_PER_W

    def part1(idx0_v, idx1_v, r0_v, r1_v, sem_g, sem_w):
        pltpu.sync_copy(idxT_hbm.at[0, pl.ds(base, B_PER_W)], idx0_v)
        pltpu.sync_copy(idxT_hbm.at[1, pl.ds(base, B_PER_W)], idx1_v)

        def load_and_fire(c, p):
            s = pl.ds(c * CHUNK, CHUNK)
            return [
                pltpu.async_copy(t0_hbm.at[idx0_v.at[s]], r0_v.at[p], sem_g),
                pltpu.async_copy(t1_hbm.at[idx1_v.at[s]], r1_v.at[p], sem_g),
            ]

        handles_g = [None, None]
        handles_w = [None, None]
        handles_g[0] = load_and_fire(0, 0)

        for c in range(NCHUNK):
            p = c % 2
            q = 1 - p
            if c + 1 < NCHUNK:
                if handles_w[q] is not None:
                    for h in handles_w[q]:
                        h.wait()
                handles_g[q] = load_and_fire(c + 1, q)
            for h in handles_g[p]:
                h.wait()

            def row_body(r, carry):
                for j in range(D // L):
                    s = pl.ds(j * L, L)
                    r0_v[p, r, s] = r0_v[p, r, s] + r1_v[p, r, s]
                return carry

            lax.fori_loop(0, CHUNK, row_body, 0)

            off = base + c * CHUNK
            handles_w[p] = [
                pltpu.async_copy(r0_v.at[p], out0_hbm.at[pl.ds(off, CHUNK)],
                                 sem_w),
            ]

        for hs in handles_w:
            if hs is not None:
                for h in hs:
                    h.wait()

    pl.run_scoped(
        part1,
        pltpu.VMEM((B_PER_W,), jnp.int32),
        pltpu.VMEM((B_PER_W,), jnp.int32),
        pltpu.VMEM((2, CHUNK, D), jnp.float32),
        pltpu.VMEM((2, CHUNK, D), jnp.float32),
        pltpu.SemaphoreType.DMA,
        pltpu.SemaphoreType.DMA,
    )

    def part2(row_v, idx_v, outc_v, sem_r, sem_w):
        nrows = lax.select(wid < EXTRA, ROWS_LO + 1, ROWS_LO)
        start = lax.select(wid < EXTRA, (ROWS_LO + 1) * wid,
                           EXTRA + ROWS_LO * wid)
        pltpu.sync_copy(idxT_hbm.at[1], idx_v)
        pltpu.async_copy(t2T_hbm.at[start], row_v, sem_r)

        def row_loop(i, carry):
            c = start + i
            pltpu.make_async_copy(t2T_hbm.at[c], row_v, sem_r).wait()
            for bc in range(BATCH // BC):
                p = bc % 2

                @pl.when(i * (BATCH // BC) + bc > 1)
                def _drain():
                    pltpu.make_async_copy(
                        outc_v.at[p],
                        out1T_hbm.at[c, pl.ds(bc * BC, BC)], sem_w).wait()

                def g_loop(g, carry2):
                    for u in range(16):
                        s = pl.ds(bc * BC + (g * 16 + u) * L, L)
                        so = pl.ds((g * 16 + u) * L, L)
                        outc_v[p, so] = plsc.load_gather(row_v, [idx_v[s]])
                    return carry2

                lax.fori_loop(0, BC // (16 * L), g_loop, 0)
                pltpu.async_copy(outc_v.at[p],
                                 out1T_hbm.at[c, pl.ds(bc * BC, BC)], sem_w)

            @pl.when(i + 1 < nrows)
            def _prefetch():
                pltpu.async_copy(t2T_hbm.at[c + 1], row_v, sem_r)

            return carry

        lax.fori_loop(0, nrows, row_loop, 0)
        last = start + nrows - 1
        for bc in range(BATCH // BC - 2, BATCH // BC):
            pltpu.make_async_copy(
                outc_v.at[bc % 2],
                out1T_hbm.at[last, pl.ds(bc * BC, BC)], sem_w).wait()

    pl.run_scoped(
        part2,
        pltpu.VMEM((VOCAB,), jnp.float32),
        pltpu.VMEM((BATCH,), jnp.int32),
        pltpu.VMEM((2, BC), jnp.float32),
        pltpu.SemaphoreType.DMA,
        pltpu.SemaphoreType.DMA,
    )


@jax.jit
def _run(t0, t1, t2T, idxT):
    mesh = plsc.VectorSubcoreMesh(core_axis_name="c", subcore_axis_name="s")
    fn = functools.partial(
        pl.kernel, mesh=mesh,
        compiler_params=pltpu.CompilerParams(needs_layout_passes=False),
        out_type=[
            jax.ShapeDtypeStruct((BATCH, D), jnp.float32),
            jax.ShapeDtypeStruct((D2, BATCH), jnp.float32),
        ],
    )(_sc_kernel)
    return fn(t0, t1, t2T, idxT)


def kernel(arg0_1, arg1_1, arg2_1, arg3_1):
    t2T = arg2_1.T
    idxT = arg3_1.astype(jnp.int32).T
    out0, out1T = _run(arg0_1, arg1_1, t2T, idxT)
    return (out0, out1T.T)
